# baseline (speedup 1.0000x reference)
import functools

import jax
import jax.numpy as jnp
from jax import lax
from jax.experimental import pallas as pl
from jax.experimental.pallas import tpu as pltpu
from jax.experimental.pallas import tpu_sc as plsc

_L = 16
_NC = 2
_NS = 16


def _sc_avg_body(slab_hbm, avg_hbm, buf, acc):
    S = avg_hbm.shape[0]
    C = avg_hbm.shape[1]
    nw = _NC * _NS
    wid = lax.axis_index("s") * _NC + lax.axis_index("c")
    scale = 1.0 / S

    for t in range((C + nw - 1) // nw):
        chunk = wid + t * nw

        @pl.when(chunk < C)
        def _():
            pltpu.sync_copy(slab_hbm.at[:, chunk, :], buf)

            def _init(k, w):
                return w + buf[(S - 1) + k, :]

            w0 = lax.fori_loop(
                0, S, _init, jnp.zeros((_L,), jnp.float32), unroll=8
            )
            acc[0, :] = w0 * scale

            def _slide(i, w):
                w = w + buf[(S - 1) - i, :] - buf[(2 * S - 1) - i, :]
                acc[i, :] = w * scale
                return w

            lax.fori_loop(1, S, _slide, w0, unroll=4)
            pltpu.sync_copy(acc, avg_hbm.at[:, chunk, :])


def _add_body(avg_ref, x_ref, o_ref):
    o_ref[...] = x_ref[...] + avg_ref[...][None]


def kernel(x, rel_table):
    B, S, D = x.shape
    C = D // _L
    max_len = (rel_table.shape[0] + 1) // 2
    lo = max_len - S
    slab = lax.slice(rel_table, (lo, 0), (lo + 2 * S - 1, D))
    slab = jnp.pad(slab, ((0, 1), (0, 0))).reshape(2 * S, C, _L)

    sc_avg = pl.kernel(
        _sc_avg_body,
        out_type=jax.ShapeDtypeStruct((S, C, _L), jnp.float32),
        mesh=plsc.VectorSubcoreMesh(core_axis_name="c", subcore_axis_name="s"),
        scratch_types=[
            pltpu.VMEM((2 * S, _L), jnp.float32),
            pltpu.VMEM((S, _L), jnp.float32),
        ],
        compiler_params=pltpu.CompilerParams(use_tc_tiling_on_sc=False),
    )
    avg = sc_avg(slab).reshape(S, D)

    return pl.pallas_call(
        _add_body,
        grid=(B,),
        in_specs=[
            pl.BlockSpec((S, D), lambda b: (0, 0)),
            pl.BlockSpec((1, S, D), lambda b: (b, 0, 0)),
        ],
        out_specs=pl.BlockSpec((1, S, D), lambda b: (b, 0, 0)),
        out_shape=jax.ShapeDtypeStruct((B, S, D), jnp.float32),
    )(avg, x)
